# Initial kernel scaffold; baseline (speedup 1.0000x reference)
import jax
import jax.numpy as jnp
from jax.experimental import pallas as pl


def kernel(x, edge_indices, edge_types, W, att_src, att_dst, bias, sem_W1, sem_b1, sem_W2):
    raise NotImplementedError("write your pallas kernel here")


# speedup vs baseline: 4.5519x; 4.5519x over previous
import functools

import jax
import jax.numpy as jnp
from jax import lax
from jax.experimental import pallas as pl
from jax.experimental.pallas import tpu as pltpu, tpu_sc as plsc

N = 10000
D = 128
E = 320000
R = 4
H = 8
O = 16
SEM_HID = 256

NC = 2
NS = 16
NW = NC * NS
PER_TILE = E // NW
C = 80
CHUNKS = PER_TILE // C
NPAD = N + 1
DROWS = N // 16 + 1
ACCR = NPAD + DROWS


def _prep_body(x_ref, wt_ref, mst_ref, mdt_ref, h_out, as_out, ad_out):
    xb = x_ref[...]
    for r in range(R):
        hr = jnp.dot(xb, wt_ref[r], preferred_element_type=jnp.float32)
        h_out[r] = hr
        as_out[r] = jnp.dot(hr, mst_ref[r], preferred_element_type=jnp.float32)
        ad_out[r] = jnp.dot(hr, mdt_ref[r], preferred_element_type=jnp.float32)


def _sc_body(rel, src_hbm, dst_hbm, typ_hbm, as_hbm, ad_hbm, h_hbm, pat_hbm,
             z_hbm, acc_out, src_v, dst_v, typ_v, deff_v, deni_v, rowp_v,
             as_v, ad_v, h_v, w_v, acc_sh, sem):
    cid = lax.axis_index("c")
    sid = lax.axis_index("s")
    wid = sid * NC + cid

    @pl.when(sid == 0)
    def _():
        pltpu.sync_copy(z_hbm, acc_sh)

    plsc.subcore_barrier()

    def chunk(k, carry):
        base = wid * PER_TILE + k * C
        pltpu.sync_copy(src_hbm.at[pl.ds(base, C)], src_v)
        pltpu.sync_copy(dst_hbm.at[pl.ds(base, C)], dst_v)
        pltpu.sync_copy(typ_hbm.at[pl.ds(base, C)], typ_v)
        pltpu.async_copy(as_hbm.at[src_v], as_v, sem).wait()
        pltpu.async_copy(ad_hbm.at[dst_v], ad_v, sem).wait()
        pltpu.async_copy(h_hbm.at[src_v], h_v, sem).wait()

        for i in range(C // 16):
            sl = pl.ds(i * 16, 16)
            de = jnp.where(typ_v[sl] == rel, dst_v[sl], N)
            deff_v[sl] = de
            deni_v[sl] = NPAD + lax.shift_right_logical(de, 4)
            rowp_v[sl] = jnp.bitwise_and(de, 15)

        def wbody(c, c2):
            al = as_v[c, pl.ds(0, 16)] + ad_v[c, pl.ds(0, 16)]
            al = jnp.where(al > 0, al, al * 0.2)
            w_v[c] = jnp.exp(al)
            return c2

        lax.fori_loop(0, C, wbody, 0)

        pltpu.async_copy(pat_hbm.at[rowp_v], ad_v, sem).wait()

        def gbody(c, c2):
            w = w_v[c]
            for g8 in range(H):
                sl2 = pl.ds(g8 * 16, 16)
                as_v[c, sl2] = h_v[c, sl2] * w
                ad_v[c, sl2] = ad_v[c, sl2] * w
            return c2

        lax.fori_loop(0, C, gbody, 0)
        pltpu.sync_copy(as_v, acc_sh.at[deff_v], add=True)
        pltpu.sync_copy(ad_v, acc_sh.at[deni_v], add=True)
        return carry

    lax.fori_loop(0, CHUNKS, chunk, 0)
    plsc.subcore_barrier()

    @pl.when(sid == 0)
    def _():
        pltpu.sync_copy(acc_sh, acc_out.at[cid])


def _combine_body(a00, a01, a02, a03, a10, a11, a12, a13, d0, d1, d2, d3,
                  h_ref, as_ref, ad_ref, typ_ref, exp_ref, perm_ref, w1t_ref,
                  b1_ref, w2_ref, bias_ref, out_ref):
    accs0 = (a00, a01, a02, a03)
    accs1 = (a10, a11, a12, a13)
    dens = (d0, d1, d2, d3)
    types = typ_ref[...]
    expand = exp_ref[...]
    zs = []
    scores = []
    for r in range(R):
        num = accs0[r][...] + accs1[r][...]
        den8 = jnp.sum(dens[r][...], axis=0)
        aself = as_ref[r][:, :16] + ad_ref[r][:, :16]
        aself = jnp.where(aself > 0, aself, aself * 0.2)
        wself = jnp.exp(aself)[:, :H]
        w128 = jnp.dot(wself, expand, preferred_element_type=jnp.float32)
        den128 = jnp.dot(den8 + wself, expand,
                         preferred_element_type=jnp.float32)
        outr_t = (num + w128 * h_ref[r]) / (den128 + 1e-16)
        outr = (jnp.dot(outr_t, perm_ref[...],
                        preferred_element_type=jnp.float32)
                + bias_ref[r][None, :])
        cnt = jnp.sum(jnp.where(types == r, 1, 0))
        zr = jnp.where(cnt > 0, outr, 0.0)
        sr = jnp.sum(
            jnp.tanh(jnp.dot(zr, w1t_ref[...],
                             preferred_element_type=jnp.float32) + b1_ref[...])
            * w2_ref[...], axis=1, keepdims=True)
        zs.append(zr)
        scores.append(sr)
    m = jnp.maximum(jnp.maximum(scores[0], scores[1]),
                    jnp.maximum(scores[2], scores[3]))
    es = [jnp.exp(s - m) for s in scores]
    se = es[0] + es[1] + es[2] + es[3]
    out_ref[...] = (es[0] * zs[0] + es[1] * zs[1] + es[2] * zs[2]
                    + es[3] * zs[3]) / se


def kernel(x, edge_indices, edge_types, W, att_src, att_dst, bias, sem_W1,
           sem_b1, sem_W2):
    f32 = jnp.float32
    WT = jnp.transpose(W, (0, 2, 1))
    P = jnp.concatenate([jnp.eye(H, dtype=f32), jnp.eye(H, dtype=f32)], axis=1)
    MsT = jnp.pad(jnp.einsum('rhj,hl->rhjl', att_src, P).reshape(R, D, 16),
                  ((0, 0), (0, 0), (0, D - 16)))
    MdT = jnp.pad(jnp.einsum('rhj,hl->rhjl', att_dst, P).reshape(R, D, 16),
                  ((0, 0), (0, 0), (0, D - 16)))
    li = jnp.arange(D)
    cmap = (li % H) * O + li // H
    WT = WT[:, :, cmap]
    MsT = MsT[:, cmap, :]
    MdT = MdT[:, cmap, :]
    PERM = jax.nn.one_hot(cmap, D, dtype=f32)
    expand = jnp.tile(jnp.eye(H, dtype=f32), (1, O))
    den_pat = (li[None, :] // H == jnp.arange(16)[:, None]).astype(f32)
    src = edge_indices[0]
    dst = edge_indices[1]
    typ = edge_types
    zeros = jnp.zeros((ACCR, D), f32)
    typm = edge_types.reshape(E // D, D)
    W1T = sem_W1.T
    b1r = sem_b1.reshape(1, SEM_HID)
    w2r = sem_W2.reshape(1, SEM_HID)

    grid = 10
    blk = N // grid
    h4, as_t, ad_t = pl.pallas_call(
        _prep_body,
        grid=(grid,),
        in_specs=[
            pl.BlockSpec((blk, D), lambda i: (i, 0)),
            pl.BlockSpec((R, D, D), lambda i: (0, 0, 0)),
            pl.BlockSpec((R, D, D), lambda i: (0, 0, 0)),
            pl.BlockSpec((R, D, D), lambda i: (0, 0, 0)),
        ],
        out_specs=[
            pl.BlockSpec((R, blk, D), lambda i: (0, i, 0)),
            pl.BlockSpec((R, blk, D), lambda i: (0, i, 0)),
            pl.BlockSpec((R, blk, D), lambda i: (0, i, 0)),
        ],
        out_shape=[
            jax.ShapeDtypeStruct((R, N, D), f32),
            jax.ShapeDtypeStruct((R, N, D), f32),
            jax.ShapeDtypeStruct((R, N, D), f32),
        ],
    )(x, WT, MsT, MdT)

    mesh = plsc.VectorSubcoreMesh(core_axis_name="c", subcore_axis_name="s")
    accs = []
    for r in range(R):
        sc = functools.partial(
            pl.kernel,
            mesh=mesh,
            out_type=jax.ShapeDtypeStruct((NC, ACCR, D), f32),
            scratch_types=[
                pltpu.VMEM((C,), jnp.int32),
                pltpu.VMEM((C,), jnp.int32),
                pltpu.VMEM((C,), jnp.int32),
                pltpu.VMEM((C,), jnp.int32),
                pltpu.VMEM((C,), jnp.int32),
                pltpu.VMEM((C,), jnp.int32),
                pltpu.VMEM((C, D), f32),
                pltpu.VMEM((C, D), f32),
                pltpu.VMEM((C, D), f32),
                pltpu.VMEM((C, 16), f32),
                pltpu.VMEM_SHARED((ACCR, D), f32),
                pltpu.SemaphoreType.DMA,
            ],
        )(functools.partial(_sc_body, r))
        accs.append(sc(src, dst, typ, as_t[r], ad_t[r], h4[r], den_pat,
                       zeros))

    a0 = [a[0, :N] for a in accs]
    a1 = [a[1, :N] for a in accs]
    d4 = [a[:, NPAD:NPAD + N // 16, :].reshape(NC, N, H) for a in accs]

    full = lambda shape: pl.BlockSpec(shape, lambda i: tuple(0 for _ in shape))
    out = pl.pallas_call(
        _combine_body,
        grid=(grid,),
        in_specs=(
            [pl.BlockSpec((blk, D), lambda i: (i, 0))] * 8
            + [pl.BlockSpec((NC, blk, H), lambda i: (0, i, 0))] * 4
            + [
                pl.BlockSpec((R, blk, D), lambda i: (0, i, 0)),
                pl.BlockSpec((R, blk, D), lambda i: (0, i, 0)),
                pl.BlockSpec((R, blk, D), lambda i: (0, i, 0)),
                full((E // D, D)),
                full((H, D)),
                full((D, D)),
                full((D, SEM_HID)),
                full((1, SEM_HID)),
                full((1, SEM_HID)),
                full((R, D)),
            ]
        ),
        out_specs=pl.BlockSpec((blk, D), lambda i: (i, 0)),
        out_shape=jax.ShapeDtypeStruct((N, D), f32),
    )(*a0, *a1, *d4, h4, as_t, ad_t, typm, expand, PERM, W1T, b1r, w2r, bias)
    return out
